# baseline (speedup 1.0000x reference)
import jax
import jax.numpy as jnp
from jax.experimental import pallas as pl
from jax.experimental.pallas import tpu as pltpu

_ROW_CHUNK = 128


def _meta_body(ds_r_ref, ds_c_ref, seg_r_ref, seg_c_ref,
               ex_ref, st_ref, en_ref):
    b = ds_r_ref.shape[1]
    ds_r = ds_r_ref[...]
    ds_c = ds_c_ref[...]
    seg_r = seg_r_ref[...]
    seg_c = seg_c_ref[...]
    big = jnp.int32(1 << 20)

    bi_c = jax.lax.broadcasted_iota(jnp.int32, (b, seg_r.shape[1]), 0)
    starts_c = jnp.sum((seg_r < bi_c).astype(jnp.int32), axis=1, keepdims=True)
    ends_c = jnp.sum((seg_r <= bi_c).astype(jnp.int32), axis=1, keepdims=True)
    bi_r = jax.lax.broadcasted_iota(jnp.int32, (seg_c.shape[0], b), 1)
    starts_r = jnp.sum((seg_c < bi_r).astype(jnp.int32), axis=0, keepdims=True)
    ends_r = jnp.sum((seg_c <= bi_r).astype(jnp.int32), axis=0, keepdims=True)

    empty_r = (ends_r == starts_r)
    empty_c = (ends_c == starts_c)
    key_r = ds_r + jnp.where(empty_r, big, 0)
    key_c = ds_c + jnp.where(empty_c, big, 0)

    ii = jax.lax.broadcasted_iota(jnp.int32, (b, b), 0)
    jj = jax.lax.broadcasted_iota(jnp.int32, (b, b), 1)
    less = key_r < key_c
    eq_before = (key_r == key_c) & (jj < ii)
    rank_c = jnp.sum((less | eq_before).astype(jnp.int32), axis=1,
                     keepdims=True)

    onehot = (rank_c == jj).astype(jnp.int32)
    ex_s = jnp.sum(onehot * ds_c, axis=0, keepdims=True)
    st_s = jnp.sum(onehot * starts_c, axis=0, keepdims=True)
    en_s = jnp.sum(onehot * ends_c, axis=0, keepdims=True)
    emp_s = jnp.sum(onehot * empty_c.astype(jnp.int32), axis=0, keepdims=True)

    p_r = jax.lax.broadcasted_iota(jnp.int32, (1, b), 1)
    n_nonempty = jnp.sum((~empty_r).astype(jnp.int32))
    fill = jnp.sum(jnp.where(p_r == n_nonempty - 1, ex_s, 0))
    ex_ref[...] = jnp.where(emp_s > 0, fill, ex_s)
    st_ref[...] = st_s
    en_ref[...] = en_s


def _moe_body(ex_ref, st_ref, en_ref, x_ref, w_ref, b_ref, out_ref):
    g = pl.program_id(0)
    start = st_ref[0, g]
    end = en_ref[0, g]
    rc = _ROW_CHUNK
    n = x_ref.shape[0]
    w = w_ref[0]
    bias = b_ref[...]
    base = (start // rc) * rc
    nch = jnp.where(end > start, (end - base + rc - 1) // rc, 0)

    def chunk(k, carry):
        cs = jnp.minimum(base + k * rc, n - rc)
        xb = x_ref[pl.ds(cs, rc), :]
        contrib = jax.lax.dot_general(
            xb, w, (((1,), (1,)), ((), ())),
            preferred_element_type=jnp.float32) + bias
        grow = cs + jax.lax.broadcasted_iota(jnp.int32, contrib.shape, 0)
        mask = (grow >= start) & (grow < end)
        cur = out_ref[pl.ds(cs, rc), :]
        out_ref[pl.ds(cs, rc), :] = jnp.where(mask, contrib, cur)
        return carry

    jax.lax.fori_loop(0, nch, chunk, 0, unroll=False)


def kernel(x, weights, bias, dataset_ids, segment_ids):
    n, d_in = x.shape
    e, d_out, _ = weights.shape
    b_count = dataset_ids.shape[0]

    ds32 = dataset_ids.astype(jnp.int32)
    seg32 = segment_ids.astype(jnp.int32)
    meta_shape = jax.ShapeDtypeStruct((1, b_count), jnp.int32)
    ex, st, en = pl.pallas_call(
        _meta_body,
        out_shape=(meta_shape, meta_shape, meta_shape),
    )(ds32.reshape(1, b_count), ds32.reshape(b_count, 1),
      seg32.reshape(1, n), seg32.reshape(n, 1))

    bias2d = bias.reshape(1, d_out)
    grid_spec = pltpu.PrefetchScalarGridSpec(
        num_scalar_prefetch=3,
        grid=(b_count,),
        in_specs=[
            pl.BlockSpec((n, d_in), lambda g, ex_r, *_: (0, 0)),
            pl.BlockSpec((1, d_out, d_in),
                         lambda g, ex_r, *_: (ex_r[0, g], 0, 0)),
            pl.BlockSpec((1, d_out), lambda g, *_: (0, 0)),
        ],
        out_specs=pl.BlockSpec((n, d_out), lambda g, *_: (0, 0)),
    )
    return pl.pallas_call(
        _moe_body,
        grid_spec=grid_spec,
        out_shape=jax.ShapeDtypeStruct((n, d_out), jnp.float32),
    )(ex, st, en, x, weights, bias2d)


# speedup vs baseline: 1.0078x; 1.0078x over previous
import jax
import jax.numpy as jnp
from jax.experimental import pallas as pl
from jax.experimental.pallas import tpu as pltpu

_ROW_CHUNK = 128


def _fused_body(ds_r_ref, ds_c_ref, seg_r_ref, seg_c_ref, x_ref, w_hbm, b_ref,
                out_ref, w_buf, meta_v, meta_s, sem_w, sem_m):
    b = ds_r_ref.shape[1]
    n = x_ref.shape[0]
    rc = _ROW_CHUNK
    ds_r = ds_r_ref[...]
    ds_c = ds_c_ref[...]
    seg_r = seg_r_ref[...]
    seg_c = seg_c_ref[...]
    big = jnp.int32(1 << 20)

    bi_c = jax.lax.broadcasted_iota(jnp.int32, (b, seg_r.shape[1]), 0)
    starts_c = jnp.sum((seg_r < bi_c).astype(jnp.int32), axis=1, keepdims=True)
    ends_c = jnp.sum((seg_r <= bi_c).astype(jnp.int32), axis=1, keepdims=True)
    bi_r = jax.lax.broadcasted_iota(jnp.int32, (seg_c.shape[0], b), 1)
    starts_r = jnp.sum((seg_c < bi_r).astype(jnp.int32), axis=0, keepdims=True)
    ends_r = jnp.sum((seg_c <= bi_r).astype(jnp.int32), axis=0, keepdims=True)

    empty_r = (ends_r == starts_r)
    empty_c = (ends_c == starts_c)
    key_r = ds_r + jnp.where(empty_r, big, 0)
    key_c = ds_c + jnp.where(empty_c, big, 0)

    ii = jax.lax.broadcasted_iota(jnp.int32, (b, b), 0)
    jj = jax.lax.broadcasted_iota(jnp.int32, (b, b), 1)
    less = key_r < key_c
    eq_before = (key_r == key_c) & (jj < ii)
    rank_c = jnp.sum((less | eq_before).astype(jnp.int32), axis=1,
                     keepdims=True)

    onehot = (rank_c == jj).astype(jnp.int32)
    ex_s = jnp.sum(onehot * ds_c, axis=0, keepdims=True)
    st_s = jnp.sum(onehot * starts_c, axis=0, keepdims=True)
    en_s = jnp.sum(onehot * ends_c, axis=0, keepdims=True)
    emp_s = jnp.sum(onehot * empty_c.astype(jnp.int32), axis=0, keepdims=True)

    p_r = jax.lax.broadcasted_iota(jnp.int32, (1, b), 1)
    n_nonempty = jnp.sum((~empty_r).astype(jnp.int32))
    fill = jnp.sum(jnp.where(p_r == n_nonempty - 1, ex_s, 0))
    meta_v[0:1, :] = jnp.where(emp_s > 0, fill, ex_s)
    meta_v[1:2, :] = st_s
    meta_v[2:3, :] = en_s
    meta_v[3:8, :] = jnp.zeros((5, b), jnp.int32)

    mcopy = pltpu.make_async_copy(meta_v, meta_s, sem_m)
    mcopy.start()
    mcopy.wait()

    bias = b_ref[...]

    e0 = meta_s[0, 0]
    pltpu.make_async_copy(w_hbm.at[e0], w_buf.at[0], sem_w.at[0]).start()

    def slot(i, carry):
        cur_buf, prev_ex = carry
        e_i = meta_s[0, i]
        e_nxt = meta_s[0, jnp.minimum(i + 1, b - 1)]
        issue = jnp.logical_and(i + 1 < b, e_nxt != e_i)
        nxt_buf = jnp.where(issue, 1 - cur_buf, cur_buf)

        @pl.when(issue)
        def _issue():
            pltpu.make_async_copy(
                w_hbm.at[e_nxt], w_buf.at[1 - cur_buf],
                sem_w.at[1 - cur_buf]).start()

        @pl.when(jnp.logical_or(i == 0, e_i != prev_ex))
        def _wait():
            pltpu.make_async_copy(
                w_hbm.at[e_i], w_buf.at[cur_buf], sem_w.at[cur_buf]).wait()

        start = meta_s[1, i]
        end = meta_s[2, i]
        base = (start // rc) * rc
        nch = jnp.where(end > start, (end - base + rc - 1) // rc, 0)
        w = w_buf[cur_buf]

        def chunk(k, carry2):
            cs = jnp.minimum(base + k * rc, n - rc)
            xb = x_ref[pl.ds(cs, rc), :]
            contrib = jax.lax.dot_general(
                xb, w, (((1,), (1,)), ((), ())),
                preferred_element_type=jnp.float32) + bias
            grow = cs + jax.lax.broadcasted_iota(jnp.int32, contrib.shape, 0)
            mask = (grow >= start) & (grow < end)
            cur = out_ref[pl.ds(cs, rc), :]
            out_ref[pl.ds(cs, rc), :] = jnp.where(mask, contrib, cur)
            return carry2

        jax.lax.fori_loop(0, nch, chunk, 0, unroll=False)
        return (nxt_buf, e_i)

    jax.lax.fori_loop(0, b, slot, (jnp.int32(0), jnp.int32(-1)),
                      unroll=False)


def kernel(x, weights, bias, dataset_ids, segment_ids):
    n, d_in = x.shape
    e, d_out, _ = weights.shape
    b_count = dataset_ids.shape[0]

    ds32 = dataset_ids.astype(jnp.int32)
    seg32 = segment_ids.astype(jnp.int32)

    return pl.pallas_call(
        _fused_body,
        in_specs=[
            pl.BlockSpec(memory_space=pltpu.MemorySpace.VMEM),
            pl.BlockSpec(memory_space=pltpu.MemorySpace.VMEM),
            pl.BlockSpec(memory_space=pltpu.MemorySpace.VMEM),
            pl.BlockSpec(memory_space=pltpu.MemorySpace.VMEM),
            pl.BlockSpec(memory_space=pltpu.MemorySpace.VMEM),
            pl.BlockSpec(memory_space=pltpu.MemorySpace.HBM),
            pl.BlockSpec(memory_space=pltpu.MemorySpace.VMEM),
        ],
        out_specs=pl.BlockSpec(memory_space=pltpu.MemorySpace.VMEM),
        out_shape=jax.ShapeDtypeStruct((n, d_out), jnp.float32),
        scratch_shapes=[
            pltpu.VMEM((2, d_out, d_in), jnp.float32),
            pltpu.VMEM((8, b_count), jnp.int32),
            pltpu.SMEM((8, b_count), jnp.int32),
            pltpu.SemaphoreType.DMA((2,)),
            pltpu.SemaphoreType.DMA,
        ],
    )(ds32.reshape(1, b_count), ds32.reshape(b_count, 1),
      seg32.reshape(1, n), seg32.reshape(n, 1),
      x, weights, bias.reshape(1, d_out))
